# baseline (speedup 1.0000x reference)
import functools

import jax
import jax.numpy as jnp
from jax import lax
from jax.experimental import pallas as pl
from jax.experimental.pallas import tpu as pltpu
from jax.experimental.pallas import tpu_sc as plsc

NUM_VOCAB = 1000
EMBED = 128
RESOLUTION = 32
B = 1024
L = 200
N = B * L

ND = RESOLUTION + 1
NP = 2 * RESOLUTION

LANES = 16
NC = 2
NS = 16
NW = NC * NS

ROWS_PER_W = N // NW
CHUNK = 128
NCHUNK = ROWS_PER_W // CHUNK

_mesh = plsc.VectorSubcoreMesh(core_axis_name="c", subcore_axis_name="s")


def _build_tables_body(dt_ref, p0_ref, p1_ref, p2_ref, dp2_ref, p01_ref):
    dp2 = dt_ref[...][:, None, :] + p2_ref[...][None, :, :]
    dp2_ref[...] = dp2.reshape(ND * NP, EMBED)
    p01 = p0_ref[...][:, None, :] + p1_ref[...][None, :, :]
    p01_ref[...] = p01.reshape(NP * NP, EMBED)


def _build_tables(dt, p0, p1, p2):
    return pl.pallas_call(
        _build_tables_body,
        out_shape=(
            jax.ShapeDtypeStruct((ND * NP, EMBED), jnp.float32),
            jax.ShapeDtypeStruct((NP * NP, EMBED), jnp.float32),
        ),
    )(dt, p0, p1, p2)


@functools.partial(
    pl.kernel,
    out_type=jax.ShapeDtypeStruct((N, EMBED), jnp.float32),
    mesh=_mesh,
    scratch_types=[
        pltpu.VMEM((2, 4, CHUNK), jnp.int32),
        pltpu.VMEM((2, CHUNK), jnp.int32),
        pltpu.VMEM((2, CHUNK), jnp.int32),
        pltpu.VMEM((2, CHUNK), jnp.int32),
        pltpu.VMEM((2, CHUNK, EMBED), jnp.float32),
        pltpu.VMEM((2, CHUNK, EMBED), jnp.float32),
        pltpu.VMEM((2, CHUNK, EMBED), jnp.float32),
        pltpu.SemaphoreType.DMA((2,)),
        pltpu.SemaphoreType.DMA((2,)),
        pltpu.SemaphoreType.DMA((2,)),
    ],
)
def _embed_sum_kernel(vidx_hbm, sidx_hbm, vtab_hbm, dp2_hbm, p01_hbm, out_hbm,
                      sidx_v, vidx_v, idp2_v, ip01_v, bufa, bufb, bufc,
                      semi, semg, semo):
    wid = lax.axis_index("s") * NC + lax.axis_index("c")

    def start_idx_dma(k, slot):
        gk = wid * NCHUNK + k
        pltpu.async_copy(vidx_hbm.at[pl.ds(gk * CHUNK, CHUNK)],
                         vidx_v.at[slot], semi.at[slot])
        pltpu.async_copy(sidx_hbm.at[gk], sidx_v.at[slot], semi.at[slot])

    def wait_idx_dma(slot):
        pltpu.make_async_copy(vidx_hbm.at[pl.ds(0, CHUNK)],
                              vidx_v.at[slot], semi.at[slot]).wait()
        pltpu.make_async_copy(sidx_hbm.at[0], sidx_v.at[slot],
                              semi.at[slot]).wait()

    def compute_idx(slot):
        for j in range(CHUNK // LANES):
            sl = pl.ds(j * LANES, LANES)
            idp2_v[slot, sl] = sidx_v[slot, 0, sl] * NP + sidx_v[slot, 3, sl]
            ip01_v[slot, sl] = sidx_v[slot, 1, sl] * NP + sidx_v[slot, 2, sl]

    def start_gathers(slot):
        pltpu.async_copy(vtab_hbm.at[vidx_v.at[slot]], bufa.at[slot],
                         semg.at[slot])
        pltpu.async_copy(dp2_hbm.at[idp2_v.at[slot]], bufb.at[slot],
                         semg.at[slot])
        pltpu.async_copy(p01_hbm.at[ip01_v.at[slot]], bufc.at[slot],
                         semg.at[slot])

    def wait_gathers(slot):
        pltpu.make_async_copy(vtab_hbm.at[vidx_v.at[slot]], bufa.at[slot],
                              semg.at[slot]).wait()
        pltpu.make_async_copy(dp2_hbm.at[idp2_v.at[slot]], bufb.at[slot],
                              semg.at[slot]).wait()
        pltpu.make_async_copy(p01_hbm.at[ip01_v.at[slot]], bufc.at[slot],
                              semg.at[slot]).wait()

    def wait_writeback(slot):
        pltpu.make_async_copy(bufa.at[slot], out_hbm.at[pl.ds(0, CHUNK)],
                              semo.at[slot]).wait()

    start_idx_dma(0, 0)
    wait_idx_dma(0)
    compute_idx(0)
    start_gathers(0)
    start_idx_dma(1, 1)

    def chunk_body(k, carry):
        cur = lax.rem(k, 2)
        nxt = 1 - cur

        @pl.when(k + 1 < NCHUNK)
        def _prep_next():
            wait_idx_dma(nxt)
            compute_idx(nxt)

            @pl.when(k >= 1)
            def _drain_prev_writeback():
                wait_writeback(nxt)

            start_gathers(nxt)

        wait_gathers(cur)

        @pl.when(k + 2 < NCHUNK)
        def _prefetch_idx():
            start_idx_dma(k + 2, cur)

        def row_body(r, carry2):
            for j in range(EMBED // LANES):
                sl = pl.ds(j * LANES, LANES)
                x = bufb[cur, r, sl] + bufc[cur, r, sl]
                plsc.addupdate(bufa.at[cur, r, sl], x)
            return carry2

        lax.fori_loop(0, CHUNK, row_body, 0)

        gk = wid * NCHUNK + k
        pltpu.async_copy(bufa.at[cur], out_hbm.at[pl.ds(gk * CHUNK, CHUNK)],
                         semo.at[cur])
        return carry

    lax.fori_loop(0, NCHUNK, chunk_body, 0)
    wait_writeback(0)
    wait_writeback(1)


def kernel(value, depth, position, value_table, depth_table, pos_table):
    vt = value_table.at[0].set(0.0)
    dp2, p01 = _build_tables(
        depth_table, pos_table[0], pos_table[1], pos_table[2]
    )

    vidx = value.reshape(N)
    sidx = jnp.stack(
        [
            depth.reshape(N),
            position[..., 0].reshape(N),
            position[..., 1].reshape(N),
            position[..., 2].reshape(N),
        ],
        axis=0,
    )
    sidx = sidx.reshape(4, N // CHUNK, CHUNK).transpose(1, 0, 2)

    out = _embed_sum_kernel(vidx, sidx, vt, dp2, p01)
    return out.reshape(B, L, EMBED)


# speedup vs baseline: 1.8326x; 1.8326x over previous
import functools

import jax
import jax.numpy as jnp
from jax import lax
from jax.experimental import pallas as pl
from jax.experimental.pallas import tpu as pltpu
from jax.experimental.pallas import tpu_sc as plsc

NUM_VOCAB = 1000
EMBED = 128
RESOLUTION = 32
B = 1024
L = 200
N = B * L

ND = RESOLUTION + 1
NP = 2 * RESOLUTION

LANES = 16
NC = 2
NS = 16
NW = NC * NS

ROWS_PER_W = N // NW
CHUNK = 128
NCHUNK = ROWS_PER_W // CHUNK

_mesh = plsc.VectorSubcoreMesh(core_axis_name="c", subcore_axis_name="s")


def _build_tables_body(dt_ref, p0_ref, p1_ref, p2_ref, dp2_ref, p01_ref):
    dp2 = dt_ref[...][:, None, :] + p2_ref[...][None, :, :]
    dp2_ref[...] = dp2.reshape(ND * NP, EMBED)
    p01 = p0_ref[...][:, None, :] + p1_ref[...][None, :, :]
    p01_ref[...] = p01.reshape(NP * NP, EMBED)


def _build_tables(dt, p0, p1, p2):
    return pl.pallas_call(
        _build_tables_body,
        out_shape=(
            jax.ShapeDtypeStruct((ND * NP, EMBED), jnp.float32),
            jax.ShapeDtypeStruct((NP * NP, EMBED), jnp.float32),
        ),
    )(dt, p0, p1, p2)


@functools.partial(
    pl.kernel,
    out_type=jax.ShapeDtypeStruct((N, EMBED), jnp.float32),
    mesh=_mesh,
    scratch_types=[
        pltpu.VMEM((2, 4, CHUNK), jnp.int32),
        pltpu.VMEM((2, CHUNK), jnp.int32),
        pltpu.VMEM((2, CHUNK), jnp.int32),
        pltpu.VMEM((2, CHUNK), jnp.int32),
        pltpu.VMEM((2, CHUNK, EMBED), jnp.float32),
        pltpu.VMEM((2, CHUNK, EMBED), jnp.float32),
        pltpu.VMEM((2, CHUNK, EMBED), jnp.float32),
        pltpu.SemaphoreType.DMA((2,)),
        pltpu.SemaphoreType.DMA((2,)),
        pltpu.SemaphoreType.DMA((2,)),
    ],
)
def _embed_sum_kernel(vidx_hbm, sidx_hbm, vtab_hbm, dp2_hbm, p01_hbm, out_hbm,
                      sidx_v, vidx_v, idp2_v, ip01_v, bufa, bufb, bufc,
                      semi, semg, semo):
    wid = lax.axis_index("s") * NC + lax.axis_index("c")

    def start_idx_dma(k, slot):
        gk = wid * NCHUNK + k
        pltpu.async_copy(vidx_hbm.at[pl.ds(gk * CHUNK, CHUNK)],
                         vidx_v.at[slot], semi.at[slot])
        pltpu.async_copy(sidx_hbm.at[gk], sidx_v.at[slot], semi.at[slot])

    def wait_idx_dma(slot):
        pltpu.make_async_copy(vidx_hbm.at[pl.ds(0, CHUNK)],
                              vidx_v.at[slot], semi.at[slot]).wait()
        pltpu.make_async_copy(sidx_hbm.at[0], sidx_v.at[slot],
                              semi.at[slot]).wait()

    def compute_idx(slot):
        for j in range(CHUNK // LANES):
            sl = pl.ds(j * LANES, LANES)
            idp2_v[slot, sl] = sidx_v[slot, 0, sl] * NP + sidx_v[slot, 3, sl]
            ip01_v[slot, sl] = sidx_v[slot, 1, sl] * NP + sidx_v[slot, 2, sl]

    def start_gathers(slot):
        pltpu.async_copy(vtab_hbm.at[vidx_v.at[slot]], bufa.at[slot],
                         semg.at[slot])
        pltpu.async_copy(dp2_hbm.at[idp2_v.at[slot]], bufb.at[slot],
                         semg.at[slot])
        pltpu.async_copy(p01_hbm.at[ip01_v.at[slot]], bufc.at[slot],
                         semg.at[slot])

    def wait_gathers(slot):
        pltpu.make_async_copy(vtab_hbm.at[vidx_v.at[slot]], bufa.at[slot],
                              semg.at[slot]).wait()
        pltpu.make_async_copy(dp2_hbm.at[idp2_v.at[slot]], bufb.at[slot],
                              semg.at[slot]).wait()
        pltpu.make_async_copy(p01_hbm.at[ip01_v.at[slot]], bufc.at[slot],
                              semg.at[slot]).wait()

    def wait_writeback(slot):
        pltpu.make_async_copy(bufa.at[slot], out_hbm.at[pl.ds(0, CHUNK)],
                              semo.at[slot]).wait()

    def sum_chunk(slot):
        @plsc.parallel_loop(0, CHUNK, unroll=2)
        def _row_body(r):
            xs = [bufb[slot, r, pl.ds(j * LANES, LANES)]
                  for j in range(EMBED // LANES)]
            ys = [bufc[slot, r, pl.ds(j * LANES, LANES)]
                  for j in range(EMBED // LANES)]
            for j in range(EMBED // LANES):
                plsc.addupdate(bufa.at[slot, r, pl.ds(j * LANES, LANES)],
                               xs[j] + ys[j])

    def half_step(k, slot):
        nxt = 1 - slot

        @pl.when(k + 1 < NCHUNK)
        def _prep_next():
            wait_idx_dma(nxt)
            compute_idx(nxt)

            @pl.when(k >= 1)
            def _drain_prev_writeback():
                wait_writeback(nxt)

            start_gathers(nxt)

        wait_gathers(slot)

        @pl.when(k + 2 < NCHUNK)
        def _prefetch_idx():
            start_idx_dma(k + 2, slot)

        sum_chunk(slot)

        gk = wid * NCHUNK + k
        pltpu.async_copy(bufa.at[slot], out_hbm.at[pl.ds(gk * CHUNK, CHUNK)],
                         semo.at[slot])

    start_idx_dma(0, 0)
    wait_idx_dma(0)
    compute_idx(0)
    start_gathers(0)
    start_idx_dma(1, 1)

    def pair_body(kk, carry):
        k = kk * 2
        half_step(k, 0)
        half_step(k + 1, 1)
        return carry

    lax.fori_loop(0, NCHUNK // 2, pair_body, 0)
    wait_writeback(0)
    wait_writeback(1)


def kernel(value, depth, position, value_table, depth_table, pos_table):
    vt = value_table.at[0].set(0.0)
    dp2, p01 = _build_tables(
        depth_table, pos_table[0], pos_table[1], pos_table[2]
    )

    vidx = value.reshape(N)
    sidx = jnp.stack(
        [
            depth.reshape(N),
            position[..., 0].reshape(N),
            position[..., 1].reshape(N),
            position[..., 2].reshape(N),
        ],
        axis=0,
    )
    sidx = sidx.reshape(4, N // CHUNK, CHUNK).transpose(1, 0, 2)

    out = _embed_sum_kernel(vidx, sidx, vt, dp2, p01)
    return out.reshape(B, L, EMBED)
